# baseline (speedup 1.0000x reference)
import functools
import math

import jax
import jax.numpy as jnp
from jax import lax
from jax.experimental import pallas as pl
from jax.experimental.pallas import tpu as pltpu
from jax.experimental.pallas import tpu_sc as plsc

NUM_VOCABS = 100000
MAX_LEN = 500
D_MODEL = 512
SL = 50
N = 1024
B = SL * N
SCALE = math.sqrt(float(D_MODEL))

LANES = 16
NW = 32
CHUNK = 64
NCHUNK = B // CHUNK
CPW = NCHUNK // NW
VPR = D_MODEL // LANES
CHUNKS_PER_SL = N // CHUNK


def _pe_table():
    position = jnp.arange(0, SL, dtype=jnp.float32)[:, None]
    div_term = 1.0 / (
        10000.0 ** (jnp.arange(0, D_MODEL, 2, dtype=jnp.float32) / D_MODEL)
    )
    pe = jnp.zeros((SL, D_MODEL), dtype=jnp.float32)
    pe = pe.at[:, 0::2].set(jnp.sin(position * div_term[None, :]))
    pe = pe.at[:, 1::2].set(jnp.cos(position * div_term[None, :]))
    return pe


_mesh = plsc.VectorSubcoreMesh(core_axis_name="c", subcore_axis_name="s")


@functools.partial(
    pl.kernel,
    mesh=_mesh,
    out_type=jax.ShapeDtypeStruct((B, D_MODEL), jnp.float32),
    scratch_types=[
        pltpu.VMEM((CHUNK,), jnp.int32),
        pltpu.VMEM((CHUNK, D_MODEL), jnp.float32),
        pltpu.VMEM((SL * D_MODEL,), jnp.float32),
        pltpu.SemaphoreType.DMA,
    ],
)
def _emb_pe_kernel(idx_hbm, emb_hbm, pe_hbm, out_hbm, idx_v, rows_v, pe_v, sem):
    wid = lax.axis_index("s") * 2 + lax.axis_index("c")
    c0 = wid * CPW

    pltpu.sync_copy(pe_hbm, pe_v)

    def chunk_body(i, carry):
        c = c0 + i
        base = c * CHUNK
        sl = c // CHUNKS_PER_SL
        pe_base = sl * D_MODEL

        pltpu.sync_copy(idx_hbm.at[pl.ds(base, CHUNK)], idx_v)
        pltpu.async_copy(emb_hbm.at[idx_v], rows_v, sem).wait()

        def row_body(r, carry2):
            for j in range(VPR):
                col = j * LANES
                v = rows_v[r, pl.ds(col, LANES)]
                pe_j = pe_v[pl.ds(pe_base + col, LANES)]
                rows_v[r, pl.ds(col, LANES)] = v * SCALE + pe_j
            return carry2

        lax.fori_loop(0, CHUNK, row_body, 0)

        pltpu.sync_copy(rows_v, out_hbm.at[pl.ds(base, CHUNK)])
        return carry

    lax.fori_loop(0, CPW, chunk_body, 0)


def kernel(x, emb):
    idx = x.reshape(-1).astype(jnp.int32)
    pe = _pe_table().reshape(-1)
    out = _emb_pe_kernel(idx, emb, pe)
    return out.reshape(SL, N, D_MODEL)


# speedup vs baseline: 2.3468x; 2.3468x over previous
import functools
import math

import jax
import jax.numpy as jnp
from jax import lax
from jax.experimental import pallas as pl
from jax.experimental.pallas import tpu as pltpu
from jax.experimental.pallas import tpu_sc as plsc

NUM_VOCABS = 100000
MAX_LEN = 500
D_MODEL = 512
SL = 50
N = 1024
B = SL * N
SCALE = math.sqrt(float(D_MODEL))

LANES = 16
NW = 32
CHUNK = 64
NCHUNK = B // CHUNK
CPW = NCHUNK // NW
VPR = D_MODEL // LANES
CHUNKS_PER_SL = N // CHUNK


def _pe_table():
    position = jnp.arange(0, SL, dtype=jnp.float32)[:, None]
    div_term = 1.0 / (
        10000.0 ** (jnp.arange(0, D_MODEL, 2, dtype=jnp.float32) / D_MODEL)
    )
    pe = jnp.zeros((SL, D_MODEL), dtype=jnp.float32)
    pe = pe.at[:, 0::2].set(jnp.sin(position * div_term[None, :]))
    pe = pe.at[:, 1::2].set(jnp.cos(position * div_term[None, :]))
    return pe


_mesh = plsc.VectorSubcoreMesh(core_axis_name="c", subcore_axis_name="s")


@functools.partial(
    pl.kernel,
    mesh=_mesh,
    out_type=jax.ShapeDtypeStruct((B, D_MODEL), jnp.float32),
    scratch_types=[
        pltpu.VMEM((CHUNK,), jnp.int32),
        pltpu.VMEM((CHUNK, D_MODEL), jnp.float32),
        pltpu.VMEM((SL * D_MODEL,), jnp.float32),
        pltpu.SemaphoreType.DMA,
    ],
)
def _emb_pe_kernel(idx_hbm, emb_hbm, pe_hbm, out_hbm, idx_v, rows_v, pe_v, sem):
    wid = lax.axis_index("s") * 2 + lax.axis_index("c")
    c0 = wid * CPW

    pltpu.sync_copy(pe_hbm, pe_v)

    def chunk_body(i, carry):
        c = c0 + i
        base = c * CHUNK
        sl = c // CHUNKS_PER_SL
        pe_base = sl * D_MODEL

        pltpu.sync_copy(idx_hbm.at[pl.ds(base, CHUNK)], idx_v)
        pltpu.async_copy(emb_hbm.at[idx_v], rows_v, sem).wait()

        pe_regs = [pe_v[pl.ds(pe_base + j * LANES, LANES)] for j in range(VPR)]

        def row_body(r, carry2):
            for j in range(VPR):
                col = j * LANES
                v = rows_v[r, pl.ds(col, LANES)]
                rows_v[r, pl.ds(col, LANES)] = v * SCALE + pe_regs[j]
            return carry2

        lax.fori_loop(0, CHUNK, row_body, 0)

        pltpu.sync_copy(rows_v, out_hbm.at[pl.ds(base, CHUNK)])
        return carry

    lax.fori_loop(0, CPW, chunk_body, 0)


def kernel(x, emb):
    idx = x.reshape(-1).astype(jnp.int32)
    pe = _pe_table().reshape(-1)
    out = _emb_pe_kernel(idx, emb, pe)
    return out.reshape(SL, N, D_MODEL)


# speedup vs baseline: 3.2144x; 1.3697x over previous
import functools
import math

import jax
import jax.numpy as jnp
from jax import lax
from jax.experimental import pallas as pl
from jax.experimental.pallas import tpu as pltpu
from jax.experimental.pallas import tpu_sc as plsc

NUM_VOCABS = 100000
MAX_LEN = 500
D_MODEL = 512
SL = 50
N = 1024
B = SL * N
SCALE = math.sqrt(float(D_MODEL))

LANES = 16
NW = 32
CHUNK = 64
NCHUNK = B // CHUNK
CPW = NCHUNK // NW
VPR = D_MODEL // LANES
CHUNKS_PER_SL = N // CHUNK


def _pe_table():
    position = jnp.arange(0, SL, dtype=jnp.float32)[:, None]
    div_term = 1.0 / (
        10000.0 ** (jnp.arange(0, D_MODEL, 2, dtype=jnp.float32) / D_MODEL)
    )
    pe = jnp.zeros((SL, D_MODEL), dtype=jnp.float32)
    pe = pe.at[:, 0::2].set(jnp.sin(position * div_term[None, :]))
    pe = pe.at[:, 1::2].set(jnp.cos(position * div_term[None, :]))
    return pe


_mesh = plsc.VectorSubcoreMesh(core_axis_name="c", subcore_axis_name="s")


NBUF = 3


@functools.partial(
    pl.kernel,
    mesh=_mesh,
    out_type=jax.ShapeDtypeStruct((B, D_MODEL), jnp.float32),
    scratch_types=(
        [pltpu.VMEM((CHUNK,), jnp.int32) for _ in range(NBUF)]
        + [pltpu.VMEM((CHUNK, D_MODEL), jnp.float32) for _ in range(NBUF)]
        + [pltpu.VMEM((SL * D_MODEL,), jnp.float32)]
        + [pltpu.SemaphoreType.DMA for _ in range(2 * NBUF)]
    ),
)
def _emb_pe_kernel(idx_hbm, emb_hbm, pe_hbm, out_hbm,
                   i0, i1, i2, r0, r1, r2, pe_v, g0, g1, g2, w0, w1, w2):
    idxb = [i0, i1, i2]
    rows = [r0, r1, r2]
    gsem = [g0, g1, g2]
    wsem = [w0, w1, w2]

    wid = lax.axis_index("s") * 2 + lax.axis_index("c")
    c0 = wid * CPW

    pltpu.sync_copy(pe_hbm, pe_v)

    def start_gather(i, b):
        base = (c0 + i) * CHUNK
        pltpu.sync_copy(idx_hbm.at[pl.ds(base, CHUNK)], idxb[b])
        return pltpu.async_copy(emb_hbm.at[idxb[b]], rows[b], gsem[b])

    def compute(i, buf):
        c = c0 + i
        pe_base = (c // CHUNKS_PER_SL) * D_MODEL
        pe_regs = [pe_v[pl.ds(pe_base + j * LANES, LANES)] for j in range(VPR)]

        def row_body(r, carry2):
            for j in range(VPR):
                col = j * LANES
                v = buf[r, pl.ds(col, LANES)]
                buf[r, pl.ds(col, LANES)] = v * SCALE + pe_regs[j]
            return carry2

        lax.fori_loop(0, CHUNK, row_body, 0)

    gd = [None] * NBUF
    wd = [None] * NBUF
    gd[0] = start_gather(0, 0)
    gd[1] = start_gather(1, 1)
    for i in range(CPW):
        b = i % NBUF
        nb = (i + 2) % NBUF
        if i + 2 < CPW:
            if wd[nb] is not None:
                wd[nb].wait()
            gd[nb] = start_gather(i + 2, nb)
        gd[b].wait()
        compute(i, rows[b])
        wd[b] = pltpu.async_copy(
            rows[b], out_hbm.at[pl.ds((c0 + i) * CHUNK, CHUNK)], wsem[b]
        )
    for b in range(NBUF):
        if wd[b] is not None:
            wd[b].wait()


def kernel(x, emb):
    idx = x.reshape(-1).astype(jnp.int32)
    pe = _pe_table().reshape(-1)
    out = _emb_pe_kernel(idx, emb, pe)
    return out.reshape(SL, N, D_MODEL)


# speedup vs baseline: 3.2990x; 1.0263x over previous
import functools
import math

import jax
import jax.numpy as jnp
from jax import lax
from jax.experimental import pallas as pl
from jax.experimental.pallas import tpu as pltpu
from jax.experimental.pallas import tpu_sc as plsc

NUM_VOCABS = 100000
MAX_LEN = 500
D_MODEL = 512
SL = 50
N = 1024
B = SL * N
SCALE = math.sqrt(float(D_MODEL))

LANES = 16
NW = 32
CHUNK = 64
NCHUNK = B // CHUNK
CPW = NCHUNK // NW
VPR = D_MODEL // LANES
CHUNKS_PER_SL = N // CHUNK


def _pe_table():
    position = jnp.arange(0, SL, dtype=jnp.float32)[:, None]
    div_term = 1.0 / (
        10000.0 ** (jnp.arange(0, D_MODEL, 2, dtype=jnp.float32) / D_MODEL)
    )
    pe = jnp.zeros((SL, D_MODEL), dtype=jnp.float32)
    pe = pe.at[:, 0::2].set(jnp.sin(position * div_term[None, :]))
    pe = pe.at[:, 1::2].set(jnp.cos(position * div_term[None, :]))
    return pe


_mesh = plsc.VectorSubcoreMesh(core_axis_name="c", subcore_axis_name="s")


NBUF = 3


@functools.partial(
    pl.kernel,
    mesh=_mesh,
    out_type=jax.ShapeDtypeStruct((B, D_MODEL), jnp.float32),
    scratch_types=(
        [pltpu.VMEM((CPW * CHUNK,), jnp.int32)]
        + [pltpu.VMEM((CHUNK, D_MODEL), jnp.float32) for _ in range(NBUF)]
        + [pltpu.VMEM((SL * D_MODEL,), jnp.float32)]
        + [pltpu.SemaphoreType.DMA for _ in range(2 * NBUF)]
    ),
)
def _emb_pe_kernel(idx_hbm, emb_hbm, pe_hbm, out_hbm,
                   idx_slab, r0, r1, r2, pe_v, g0, g1, g2, w0, w1, w2):
    rows = [r0, r1, r2]
    gsem = [g0, g1, g2]
    wsem = [w0, w1, w2]

    wid = lax.axis_index("s") * 2 + lax.axis_index("c")
    c0 = wid * CPW

    pltpu.sync_copy(idx_hbm.at[pl.ds(c0 * CHUNK, CPW * CHUNK)], idx_slab)
    pltpu.sync_copy(pe_hbm, pe_v)

    def start_gather(i, b):
        return pltpu.async_copy(
            emb_hbm.at[idx_slab.at[pl.ds(i * CHUNK, CHUNK)]], rows[b], gsem[b]
        )

    def compute(i, buf):
        c = c0 + i
        pe_base = (c // CHUNKS_PER_SL) * D_MODEL
        pe_regs = [pe_v[pl.ds(pe_base + j * LANES, LANES)] for j in range(VPR)]

        def row_body(r, carry2):
            for j in range(VPR):
                col = j * LANES
                v = buf[r, pl.ds(col, LANES)]
                buf[r, pl.ds(col, LANES)] = v * SCALE + pe_regs[j]
            return carry2

        lax.fori_loop(0, CHUNK, row_body, 0)

    gd = [None] * NBUF
    wd = [None] * NBUF
    gd[0] = start_gather(0, 0)
    gd[1] = start_gather(1, 1)
    for i in range(CPW):
        b = i % NBUF
        nb = (i + 2) % NBUF
        if i + 2 < CPW:
            if wd[nb] is not None:
                wd[nb].wait()
            gd[nb] = start_gather(i + 2, nb)
        gd[b].wait()
        compute(i, rows[b])
        wd[b] = pltpu.async_copy(
            rows[b], out_hbm.at[pl.ds((c0 + i) * CHUNK, CHUNK)], wsem[b]
        )
    for b in range(NBUF):
        if wd[b] is not None:
            wd[b].wait()


def kernel(x, emb):
    idx = x.reshape(-1).astype(jnp.int32)
    pe = _pe_table().reshape(-1)
    out = _emb_pe_kernel(idx, emb, pe)
    return out.reshape(SL, N, D_MODEL)
